# baseline (speedup 1.0000x reference)
import jax
import jax.numpy as jnp
from jax import lax
from jax.experimental import pallas as pl
from jax.experimental.pallas import tpu as pltpu
from jax.experimental.pallas import tpu_sc as plsc

_MAX_LEN = 8192
_D_MODEL = 2048
_NUM_CORES = 2
_NUM_SUBCORES = 16
_NUM_WORKERS = _NUM_CORES * _NUM_SUBCORES
_ROWS_PER_WORKER = _MAX_LEN // _NUM_WORKERS


def _pe_lookup_body(table_hbm, out_hbm):
    wid = lax.axis_index("s") * _NUM_CORES + lax.axis_index("c")
    base = wid * _ROWS_PER_WORKER
    pltpu.sync_copy(
        table_hbm.at[pl.ds(base, _ROWS_PER_WORKER)],
        out_hbm.at[pl.ds(base, _ROWS_PER_WORKER)],
    )


_pe_lookup = pl.kernel(
    _pe_lookup_body,
    out_type=jax.ShapeDtypeStruct((_MAX_LEN, _D_MODEL), jnp.float32),
    mesh=plsc.VectorSubcoreMesh(
        core_axis_name="c",
        subcore_axis_name="s",
        num_cores=_NUM_CORES,
        num_subcores=_NUM_SUBCORES,
    ),
)


def kernel(seq_len, pe_weight):
    del seq_len
    return _pe_lookup(pe_weight)[None]


# speedup vs baseline: 31.1488x; 31.1488x over previous
import jax
import jax.numpy as jnp
from jax import lax
from jax.experimental import pallas as pl
from jax.experimental.pallas import tpu as pltpu
from jax.experimental.pallas import tpu_sc as plsc

_MAX_LEN = 8192
_D_MODEL = 2048
_NUM_CORES = 2
_NUM_SUBCORES = 16
_NUM_WORKERS = _NUM_CORES * _NUM_SUBCORES
_ROWS_PER_WORKER = _MAX_LEN // _NUM_WORKERS


_CHUNK_ROWS = 16
_NUM_CHUNKS = _ROWS_PER_WORKER // _CHUNK_ROWS


def _pe_lookup_body(table_hbm, out_hbm, buf0, buf1, sem0, sem1):
    wid = lax.axis_index("s") * _NUM_CORES + lax.axis_index("c")
    base = wid * _ROWS_PER_WORKER
    bufs = (buf0, buf1)
    sems = (sem0, sem1)

    def gather(i, buf, sem):
        return pltpu.make_async_copy(
            table_hbm.at[pl.ds(base + i * _CHUNK_ROWS, _CHUNK_ROWS)], buf, sem
        )

    gather(0, bufs[0], sems[0]).start()
    for i in range(_NUM_CHUNKS):
        cur, csem = bufs[i % 2], sems[i % 2]
        if i + 1 < _NUM_CHUNKS:
            gather(i + 1, bufs[(i + 1) % 2], sems[(i + 1) % 2]).start()
        gather(i, cur, csem).wait()
        pltpu.sync_copy(cur, out_hbm.at[pl.ds(base + i * _CHUNK_ROWS, _CHUNK_ROWS)])


_pe_lookup = pl.kernel(
    _pe_lookup_body,
    out_type=jax.ShapeDtypeStruct((_MAX_LEN, _D_MODEL), jnp.float32),
    mesh=plsc.VectorSubcoreMesh(
        core_axis_name="c",
        subcore_axis_name="s",
        num_cores=_NUM_CORES,
        num_subcores=_NUM_SUBCORES,
    ),
    scratch_types=[
        pltpu.VMEM((_CHUNK_ROWS, _D_MODEL), jnp.float32),
        pltpu.VMEM((_CHUNK_ROWS, _D_MODEL), jnp.float32),
        pltpu.SemaphoreType.DMA,
        pltpu.SemaphoreType.DMA,
    ],
)


def kernel(seq_len, pe_weight):
    del seq_len
    return _pe_lookup(pe_weight)[None]
